# baseline (speedup 1.0000x reference)
import functools

import jax
import jax.numpy as jnp
from jax import lax
from jax.experimental import pallas as pl
from jax.experimental.pallas import tpu as pltpu
from jax.experimental.pallas import tpu_sc as plsc

_EPS = 1e-05
_NC = 2
_NS = 16
_LANES = 16


def _unroll_factor(n, cap=16):
    for u in range(cap, 0, -1):
        if n % u == 0:
            return u
    return 1


def _make_histogram(n_workers, chunk, n_pad):
    stripe = n_pad // _NS
    lo = int(chunk * 0.525) // (8 * _LANES) * (8 * _LANES)
    hi = chunk - lo
    ones_n = -(-hi // (_LANES * _LANES)) * (_LANES * _LANES)
    rows_lo = lo // _LANES
    u_hist = _unroll_factor(rows_lo)
    u_zero = _unroll_factor(n_pad // _LANES)
    u_ones = _unroll_factor(ones_n // _LANES)
    u_zs = _unroll_factor(stripe // _LANES)
    mesh = plsc.VectorSubcoreMesh(core_axis_name="c", subcore_axis_name="s")

    @functools.partial(
        pl.kernel,
        out_type=(
            jax.ShapeDtypeStruct((_NC, 1, n_pad), jnp.float32),
            jax.ShapeDtypeStruct((n_workers, 1, n_pad), jnp.float32),
        ),
        mesh=mesh,
        scratch_types=[
            pltpu.VMEM((lo,), jnp.int32),
            pltpu.VMEM((hi,), jnp.int32),
            pltpu.VMEM((ones_n,), jnp.float32),
            pltpu.VMEM((stripe,), jnp.float32),
            pltpu.VMEM((n_pad,), jnp.float32),
            pltpu.VMEM_SHARED((n_pad,), jnp.float32),
            pltpu.SemaphoreType.DMA,
            pltpu.SemaphoreType.DMA,
            pltpu.SemaphoreType.DMA,
        ],
        compiler_params=pltpu.CompilerParams(needs_layout_passes=False),
    )
    def hist(edges_hbm, out_sp, out_tile, idx_lo, idx_hi, ones_v, zeros_v,
             cnt_v, shared, sem_lo, sem_hi, sem_add):
        cid = lax.axis_index("c")
        sid = lax.axis_index("s")
        wid = sid * _NC + cid
        base = wid * chunk

        in_lo = pltpu.async_copy(edges_hbm.at[pl.ds(base, lo)], idx_lo, sem_lo)
        in_hi = pltpu.async_copy(edges_hbm.at[pl.ds(base + lo, hi)], idx_hi,
                                 sem_hi)

        def fill_ones(i, carry):
            for j in range(u_ones):
                ones_v[pl.ds((i * u_ones + j) * _LANES, _LANES)] = (
                    jnp.ones((_LANES,), jnp.float32))
            return carry

        lax.fori_loop(0, ones_n // _LANES // u_ones, fill_ones, 0)

        def fill_zeros(i, carry):
            for j in range(u_zero):
                cnt_v[pl.ds((i * u_zero + j) * _LANES, _LANES)] = (
                    jnp.zeros((_LANES,), jnp.float32))
            return carry

        lax.fori_loop(0, n_pad // _LANES // u_zero, fill_zeros, 0)

        def fill_zs(i, carry):
            for j in range(u_zs):
                zeros_v[pl.ds((i * u_zs + j) * _LANES, _LANES)] = (
                    jnp.zeros((_LANES,), jnp.float32))
            return carry

        lax.fori_loop(0, stripe // _LANES // u_zs, fill_zs, 0)

        pltpu.sync_copy(zeros_v, shared.at[pl.ds(sid * stripe, stripe)])
        in_hi.wait()
        plsc.subcore_barrier()
        add_hi = pltpu.async_copy(ones_v.at[pl.ds(0, hi)], shared.at[idx_hi],
                                  sem_add, add=True)

        in_lo.wait()
        ones = jnp.ones((_LANES,), jnp.float32)

        def body(i, carry):
            for j in range(u_hist):
                plsc.addupdate_scatter(
                    cnt_v, [idx_lo[pl.ds((i * u_hist + j) * _LANES, _LANES)]],
                    ones)
            return carry

        lax.fori_loop(0, rows_lo // u_hist, body, 0)
        pltpu.sync_copy(cnt_v, out_tile.at[wid, 0])

        add_hi.wait()
        plsc.subcore_barrier()
        pltpu.sync_copy(shared.at[pl.ds(sid * stripe, stripe)],
                        out_sp.at[cid, 0, pl.ds(sid * stripe, stripe)])

    return hist


def _mm_body(f_ref, w_ref, out_ref):
    x = jnp.dot(f_ref[...], w_ref[...], preferred_element_type=jnp.float32)
    d = x.shape[-1]
    n2 = jnp.dot(x * x, jnp.ones((d, 1), jnp.float32),
                 preferred_element_type=jnp.float32)
    nrm = jnp.sqrt(n2)
    out_ref[...] = (x * (nrm / (nrm + _EPS))).astype(jnp.bfloat16)


def _make_mm(n, d, blk):
    return pl.pallas_call(
        _mm_body,
        grid=(-(-n // blk),),
        in_specs=[
            pl.BlockSpec((blk, d), lambda i: (i, 0)),
            pl.BlockSpec((d, d), lambda i: (0, 0)),
        ],
        out_specs=pl.BlockSpec((blk, d), lambda i: (i, 0)),
        out_shape=jax.ShapeDtypeStruct((n, d), jnp.bfloat16),
    )


def _scale_body(y_ref, csp_ref, ct_ref, out_ref):
    cnt = csp_ref[0, 0] + csp_ref[1, 0]
    for w in range(ct_ref.shape[0]):
        cnt = cnt + ct_ref[w, 0]
    blk = cnt.shape[-1]
    s = jnp.sqrt(cnt).reshape(blk, 1)
    out_ref[...] = y_ref[...].astype(jnp.float32) * s


def _make_scale(n, d, nw, n_pad, blk):
    return pl.pallas_call(
        _scale_body,
        grid=(-(-n // blk),),
        in_specs=[
            pl.BlockSpec((blk, d), lambda i: (i, 0)),
            pl.BlockSpec((_NC, 1, blk), lambda i: (0, 0, i)),
            pl.BlockSpec((nw, 1, blk), lambda i: (0, 0, i)),
        ],
        out_specs=pl.BlockSpec((blk, d), lambda i: (i, 0)),
        out_shape=jax.ShapeDtypeStruct((n, d), jnp.float32),
    )


def kernel(node_features, edge_dst, Wq, Wk, Wv, w_dot):
    n, d = node_features.shape
    e = edge_dst.shape[0]
    nw = _NC * _NS
    assert e % (nw * _LANES) == 0
    chunk = e // nw
    n_pad = -(-n // (_NS * _LANES)) * (_NS * _LANES)

    csp, ct = _make_histogram(nw, chunk, n_pad)(edge_dst)

    blk = n_pad // 2
    y = _make_mm(n, d, blk)(node_features, Wv)
    return _make_scale(n, d, nw, n_pad, n_pad // 4)(y, csp, ct)


# speedup vs baseline: 1.0261x; 1.0261x over previous
import functools

import jax
import jax.numpy as jnp
from jax import lax
from jax.experimental import pallas as pl
from jax.experimental.pallas import tpu as pltpu
from jax.experimental.pallas import tpu_sc as plsc

_EPS = 1e-05
_NC = 2
_NS = 16
_LANES = 16


def _unroll_factor(n, cap=16):
    for u in range(cap, 0, -1):
        if n % u == 0:
            return u
    return 1


def _make_histogram(n_workers, chunk, n_pad):
    stripe = n_pad // _NS
    lo = int(chunk * 0.525) // (8 * _LANES) * (8 * _LANES)
    hi = chunk - lo
    ones_n = -(-hi // (_LANES * _LANES)) * (_LANES * _LANES)
    rows_lo = lo // _LANES
    u_hist = _unroll_factor(rows_lo)
    u_zero = _unroll_factor(n_pad // _LANES)
    u_ones = _unroll_factor(ones_n // _LANES)
    u_zs = _unroll_factor(stripe // _LANES)
    mesh = plsc.VectorSubcoreMesh(core_axis_name="c", subcore_axis_name="s")

    @functools.partial(
        pl.kernel,
        out_type=(
            jax.ShapeDtypeStruct((_NC, 1, n_pad), jnp.float32),
            jax.ShapeDtypeStruct((n_workers, 1, n_pad), jnp.float32),
        ),
        mesh=mesh,
        scratch_types=[
            pltpu.VMEM((lo,), jnp.int32),
            pltpu.VMEM((hi,), jnp.int32),
            pltpu.VMEM((ones_n,), jnp.float32),
            pltpu.VMEM((stripe,), jnp.float32),
            pltpu.VMEM((n_pad,), jnp.float32),
            pltpu.VMEM_SHARED((n_pad,), jnp.float32),
            pltpu.SemaphoreType.DMA,
            pltpu.SemaphoreType.DMA,
            pltpu.SemaphoreType.DMA,
        ],
        compiler_params=pltpu.CompilerParams(needs_layout_passes=False),
    )
    def hist(edges_hbm, out_sp, out_tile, idx_lo, idx_hi, ones_v, zeros_v,
             cnt_v, shared, sem_lo, sem_hi, sem_add):
        cid = lax.axis_index("c")
        sid = lax.axis_index("s")
        wid = sid * _NC + cid
        base = wid * chunk

        in_lo = pltpu.async_copy(edges_hbm.at[pl.ds(base, lo)], idx_lo, sem_lo)
        in_hi = pltpu.async_copy(edges_hbm.at[pl.ds(base + lo, hi)], idx_hi,
                                 sem_hi)

        def fill_ones(i, carry):
            for j in range(u_ones):
                ones_v[pl.ds((i * u_ones + j) * _LANES, _LANES)] = (
                    jnp.ones((_LANES,), jnp.float32))
            return carry

        lax.fori_loop(0, ones_n // _LANES // u_ones, fill_ones, 0)

        def fill_zeros(i, carry):
            for j in range(u_zero):
                cnt_v[pl.ds((i * u_zero + j) * _LANES, _LANES)] = (
                    jnp.zeros((_LANES,), jnp.float32))
            return carry

        lax.fori_loop(0, n_pad // _LANES // u_zero, fill_zeros, 0)

        def fill_zs(i, carry):
            for j in range(u_zs):
                zeros_v[pl.ds((i * u_zs + j) * _LANES, _LANES)] = (
                    jnp.zeros((_LANES,), jnp.float32))
            return carry

        lax.fori_loop(0, stripe // _LANES // u_zs, fill_zs, 0)

        pltpu.sync_copy(zeros_v, shared.at[pl.ds(sid * stripe, stripe)])
        in_hi.wait()
        plsc.subcore_barrier()
        add_hi = pltpu.async_copy(ones_v.at[pl.ds(0, hi)], shared.at[idx_hi],
                                  sem_add, add=True)

        in_lo.wait()
        ones = jnp.ones((_LANES,), jnp.float32)

        def body(i, carry):
            for j in range(u_hist):
                plsc.addupdate_scatter(
                    cnt_v, [idx_lo[pl.ds((i * u_hist + j) * _LANES, _LANES)]],
                    ones)
            return carry

        lax.fori_loop(0, rows_lo // u_hist, body, 0)
        pltpu.sync_copy(cnt_v, out_tile.at[wid, 0])

        add_hi.wait()
        plsc.subcore_barrier()
        pltpu.sync_copy(shared.at[pl.ds(sid * stripe, stripe)],
                        out_sp.at[cid, 0, pl.ds(sid * stripe, stripe)])

    return hist


def _mm_body(f_ref, w_ref, out_ref):
    x = jnp.dot(f_ref[...], w_ref[...], preferred_element_type=jnp.float32)
    d = x.shape[-1]
    n2 = jnp.dot(x * x, jnp.ones((d, 1), jnp.float32),
                 preferred_element_type=jnp.float32)
    nrm = jnp.sqrt(n2)
    out_ref[...] = (x * (nrm / (nrm + _EPS))).astype(jnp.bfloat16)


def _make_mm(n, d, blk):
    return pl.pallas_call(
        _mm_body,
        grid=(-(-n // blk),),
        in_specs=[
            pl.BlockSpec((blk, d), lambda i: (i, 0)),
            pl.BlockSpec((d, d), lambda i: (0, 0)),
        ],
        out_specs=pl.BlockSpec((blk, d), lambda i: (i, 0)),
        out_shape=jax.ShapeDtypeStruct((n, d), jnp.bfloat16),
    )


def _scale_body(y_ref, csp_ref, ct_ref, out_ref):
    blk = y_ref.shape[0]
    sl = pl.ds(pl.program_id(0) * blk, blk)
    cnt = csp_ref[0, 0, sl] + csp_ref[1, 0, sl]
    for w in range(ct_ref.shape[0]):
        cnt = cnt + ct_ref[w, 0, sl]
    s = jnp.sqrt(cnt).reshape(blk, 1)
    out_ref[...] = y_ref[...].astype(jnp.float32) * s


def _make_scale(n, d, nw, n_pad, blk):
    return pl.pallas_call(
        _scale_body,
        grid=(-(-n // blk),),
        in_specs=[
            pl.BlockSpec((blk, d), lambda i: (i, 0)),
            pl.BlockSpec((_NC, 1, n_pad), lambda i: (0, 0, 0)),
            pl.BlockSpec((nw, 1, n_pad), lambda i: (0, 0, 0)),
        ],
        out_specs=pl.BlockSpec((blk, d), lambda i: (i, 0)),
        out_shape=jax.ShapeDtypeStruct((n, d), jnp.float32),
    )


def kernel(node_features, edge_dst, Wq, Wk, Wv, w_dot):
    n, d = node_features.shape
    e = edge_dst.shape[0]
    nw = _NC * _NS
    assert e % (nw * _LANES) == 0
    chunk = e // nw
    n_pad = -(-n // (_NS * _LANES)) * (_NS * _LANES)

    csp, ct = _make_histogram(nw, chunk, n_pad)(edge_dst)

    blk = n_pad // 2
    y = _make_mm(n, d, blk)(node_features, Wv)
    return _make_scale(n, d, nw, n_pad, blk)(y, csp, ct)


# speedup vs baseline: 1.0336x; 1.0073x over previous
import functools

import jax
import jax.numpy as jnp
from jax import lax
from jax.experimental import pallas as pl
from jax.experimental.pallas import tpu as pltpu
from jax.experimental.pallas import tpu_sc as plsc

_EPS = 1e-05
_NC = 2
_NS = 16
_LANES = 16


def _unroll_factor(n, cap=16):
    for u in range(cap, 0, -1):
        if n % u == 0:
            return u
    return 1


def _make_histogram(n_workers, chunk, n_pad):
    stripe = n_pad // _NS
    lo = int(chunk * 0.525) // (8 * _LANES) * (8 * _LANES)
    hi = chunk - lo
    ones_n = -(-hi // (_LANES * _LANES)) * (_LANES * _LANES)
    rows_lo = lo // _LANES
    u_hist = _unroll_factor(rows_lo)
    u_zero = _unroll_factor(n_pad // _LANES)
    u_ones = _unroll_factor(ones_n // _LANES)
    u_zs = _unroll_factor(stripe // _LANES)
    mesh = plsc.VectorSubcoreMesh(core_axis_name="c", subcore_axis_name="s")

    @functools.partial(
        pl.kernel,
        out_type=(
            jax.ShapeDtypeStruct((_NC, 1, n_pad), jnp.float32),
            jax.ShapeDtypeStruct((n_workers, 1, n_pad), jnp.float32),
        ),
        mesh=mesh,
        scratch_types=[
            pltpu.VMEM((lo,), jnp.int32),
            pltpu.VMEM((hi,), jnp.int32),
            pltpu.VMEM((ones_n,), jnp.float32),
            pltpu.VMEM((stripe,), jnp.float32),
            pltpu.VMEM((n_pad,), jnp.float32),
            pltpu.VMEM_SHARED((n_pad,), jnp.float32),
            pltpu.SemaphoreType.DMA,
            pltpu.SemaphoreType.DMA,
            pltpu.SemaphoreType.DMA,
        ],
        compiler_params=pltpu.CompilerParams(needs_layout_passes=False),
    )
    def hist(edges_hbm, out_sp, out_tile, idx_lo, idx_hi, ones_v, zeros_v,
             cnt_v, shared, sem_lo, sem_hi, sem_add):
        cid = lax.axis_index("c")
        sid = lax.axis_index("s")
        wid = sid * _NC + cid
        base = wid * chunk

        in_lo = pltpu.async_copy(edges_hbm.at[pl.ds(base, lo)], idx_lo, sem_lo)
        in_hi = pltpu.async_copy(edges_hbm.at[pl.ds(base + lo, hi)], idx_hi,
                                 sem_hi)

        def fill_ones(i, carry):
            for j in range(u_ones):
                ones_v[pl.ds((i * u_ones + j) * _LANES, _LANES)] = (
                    jnp.ones((_LANES,), jnp.float32))
            return carry

        lax.fori_loop(0, ones_n // _LANES // u_ones, fill_ones, 0)

        def fill_zeros(i, carry):
            for j in range(u_zero):
                cnt_v[pl.ds((i * u_zero + j) * _LANES, _LANES)] = (
                    jnp.zeros((_LANES,), jnp.float32))
            return carry

        lax.fori_loop(0, n_pad // _LANES // u_zero, fill_zeros, 0)

        def fill_zs(i, carry):
            for j in range(u_zs):
                zeros_v[pl.ds((i * u_zs + j) * _LANES, _LANES)] = (
                    jnp.zeros((_LANES,), jnp.float32))
            return carry

        lax.fori_loop(0, stripe // _LANES // u_zs, fill_zs, 0)

        pltpu.sync_copy(zeros_v, shared.at[pl.ds(sid * stripe, stripe)])
        in_hi.wait()
        plsc.subcore_barrier()
        add_hi = pltpu.async_copy(ones_v.at[pl.ds(0, hi)], shared.at[idx_hi],
                                  sem_add, add=True)

        in_lo.wait()
        ones = jnp.ones((_LANES,), jnp.float32)

        def body(i, carry):
            for j in range(u_hist):
                plsc.addupdate_scatter(
                    cnt_v, [idx_lo[pl.ds((i * u_hist + j) * _LANES, _LANES)]],
                    ones)
            return carry

        lax.fori_loop(0, rows_lo // u_hist, body, 0)
        pltpu.sync_copy(cnt_v, out_tile.at[wid, 0])

        add_hi.wait()
        plsc.subcore_barrier()
        pltpu.sync_copy(shared.at[pl.ds(sid * stripe, stripe)],
                        out_sp.at[cid, 0, pl.ds(sid * stripe, stripe)])

    return hist


def _mm_body(f_ref, w_ref, out_ref):
    x = jnp.dot(f_ref[...], w_ref[...], preferred_element_type=jnp.float32)
    d = x.shape[-1]
    n2 = jnp.dot(x * x, jnp.ones((d, 1), jnp.float32),
                 preferred_element_type=jnp.float32)
    nrm = jnp.sqrt(n2)
    out_ref[...] = (x * (nrm / (nrm + _EPS))).astype(jnp.bfloat16)


def _make_mm(n, d, blk):
    return pl.pallas_call(
        _mm_body,
        grid=(-(-n // blk),),
        in_specs=[
            pl.BlockSpec((blk, d), lambda i: (i, 0)),
            pl.BlockSpec((d, d), lambda i: (0, 0)),
        ],
        out_specs=pl.BlockSpec((blk, d), lambda i: (i, 0)),
        out_shape=jax.ShapeDtypeStruct((n, d), jnp.bfloat16),
    )


def _scale_body(y_ref, csp_ref, ct_ref, out_ref):
    cnt = csp_ref[0, 0] + csp_ref[1, 0]
    for w in range(ct_ref.shape[0]):
        cnt = cnt + ct_ref[w, 0]
    blk = cnt.shape[-1]
    s = jnp.sqrt(cnt).reshape(blk, 1)
    out_ref[...] = y_ref[...].astype(jnp.float32) * s


def _make_scale(n, d, nw, n_pad, blk):
    return pl.pallas_call(
        _scale_body,
        grid=(-(-n // blk),),
        in_specs=[
            pl.BlockSpec((blk, d), lambda i: (i, 0)),
            pl.BlockSpec((_NC, 1, blk), lambda i: (0, 0, i)),
            pl.BlockSpec((nw, 1, blk), lambda i: (0, 0, i)),
        ],
        out_specs=pl.BlockSpec((blk, d), lambda i: (i, 0)),
        out_shape=jax.ShapeDtypeStruct((n, d), jnp.float32),
    )


def kernel(node_features, edge_dst, Wq, Wk, Wv, w_dot):
    n, d = node_features.shape
    e = edge_dst.shape[0]
    nw = _NC * _NS
    assert e % (nw * _LANES) == 0
    chunk = e // nw
    n_pad = -(-n // (_NS * _LANES)) * (_NS * _LANES)

    csp, ct = _make_histogram(nw, chunk, n_pad)(edge_dst)

    blk = n_pad // 2
    y = _make_mm(n, d, blk)(node_features, Wv)
    return _make_scale(n, d, nw, n_pad, blk)(y, csp, ct)
